# baseline (speedup 1.0000x reference)
import jax
import jax.numpy as jnp
from jax import lax
from jax.experimental import pallas as pl
from jax.experimental.pallas import tpu as pltpu
from jax.experimental.pallas import tpu_sc as plsc

NUM_CLASSES = 1000000
EMBED = 64
BATCH = 4096
SEQ = 200

NC = 2
NS = 16
NW = NC * NS
ROWS_PER_W = BATCH // NW

RB = 128
NBLK = (NUM_CLASSES + RB - 1) // RB
PAD_ROWS = NBLK * RB
ITERS_1 = (NBLK + NW - 1) // NW


def _tbody(tabT, tabR, in2, out2, si0, si1, so0, so1):
    wid = lax.axis_index("s") * NC + lax.axis_index("c")
    si = (si0, si1)
    so = (so0, so1)

    def start_in(blk, p):
        pltpu.make_async_copy(
            tabT.at[:, pl.ds(blk * RB, RB)], in2.at[p], si[p]
        ).start()

    def wait_in(p):
        pltpu.make_async_copy(
            tabT.at[:, pl.ds(0, RB)], in2.at[p], si[p]
        ).wait()

    def start_out(blk, p):
        pltpu.make_async_copy(
            out2.at[p], tabR.at[pl.ds(blk * RB, RB)], so[p]
        ).start()

    def wait_out(p):
        pltpu.make_async_copy(
            out2.at[p], tabR.at[pl.ds(0, RB)], so[p]
        ).wait()

    eye = lax.iota(jnp.int32, 16)

    def transpose_blk(p):
        @plsc.parallel_loop(0, RB, step=1, unroll=4)
        def _(c):
            col = jnp.full((16,), c, jnp.int32)
            for eb in range(EMBED // 16):
                v = plsc.load_gather(in2.at[p], [eye + (eb * 16), col])
                plsc.store_scatter(out2.at[p], [col, eye + (eb * 16)], v)

    start_in(wid, 0)

    @pl.when(wid + NW < NBLK)
    def _():
        start_in(wid + NW, 1)

    def step(i, carry):
        for p in range(2):
            blk = wid + (2 * i + p) * NW
            nxt = blk + 2 * NW

            @pl.when(blk < NBLK)
            def _():
                wait_in(p)

                @pl.when(blk >= 2 * NW)
                def _():
                    wait_out(p)

                transpose_blk(p)

                @pl.when(nxt < NBLK)
                def _():
                    start_in(nxt, p)

                start_out(blk, p)
        return carry

    lax.fori_loop(0, (ITERS_1 + 1) // 2, step, 0)
    wait_out(0)
    wait_out(1)


CHUNKS = ((0, 128), (128, 72))
NG = 2
NO = 2


def _gbody(x_hbm, tabR, out_hbm, idx_all, rows, outb, sg0, sg1, so0, so1):
    wid = lax.axis_index("s") * NC + lax.axis_index("c")
    row0 = wid * ROWS_PER_W
    sg = (sg0, sg1)
    so = (so0, so1)

    pltpu.sync_copy(x_hbm.at[pl.ds(row0 * SEQ, ROWS_PER_W * SEQ)], idx_all)

    def start_gather(r, p):
        base = r * SEQ
        for off, n in CHUNKS:
            pltpu.make_async_copy(
                tabR.at[idx_all.at[pl.ds(base + off, n)]],
                rows.at[p].at[pl.ds(off, n)],
                sg[p],
            ).start()

    def wait_gather(p):
        for off, n in CHUNKS:
            pltpu.make_async_copy(
                tabR.at[pl.ds(0, n)],
                rows.at[p].at[pl.ds(off, n)],
                sg[p],
            ).wait()

    eye = lax.iota(jnp.int32, 16)

    def transpose(p, q):
        @plsc.parallel_loop(0, SEQ, step=1, unroll=4)
        def _(l):
            col = jnp.full((16,), l, jnp.int32)
            for eb in range(EMBED // 16):
                v = rows.at[p][l, pl.ds(eb * 16, 16)]
                plsc.store_scatter(outb.at[q], [eye + (eb * 16), col], v)

    def start_store(r, q):
        pltpu.make_async_copy(outb.at[q], out_hbm.at[row0 + r], so[q]).start()

    def wait_store(q):
        pltpu.make_async_copy(outb.at[q], out_hbm.at[row0], so[q]).wait()

    start_gather(0, 0)
    start_gather(1, 1)

    def step(k, carry):
        for j in range(2):
            r = 2 * k + j
            p = j
            q = j

            wait_gather(p)

            @pl.when(r >= NO)
            def _():
                wait_store(q)

            transpose(p, q)

            @pl.when(r + 2 < ROWS_PER_W)
            def _():
                start_gather(r + 2, p)

            start_store(r, q)
        return carry

    lax.fori_loop(0, ROWS_PER_W // 2, step, 0)
    wait_store(0)
    wait_store(1)


def _mesh():
    return plsc.VectorSubcoreMesh(
        core_axis_name="c", subcore_axis_name="s", num_cores=NC, num_subcores=NS
    )


_PARAMS = pltpu.CompilerParams(
    use_tc_tiling_on_sc=True, needs_layout_passes=False
)
_PARAMS_LINEAR = pltpu.CompilerParams(
    use_tc_tiling_on_sc=False, needs_layout_passes=False
)


_DEBUG_XLA_GATHER = False


@jax.jit
def _run(x, embed_table):
    t = pl.kernel(
        _tbody,
        out_type=jax.ShapeDtypeStruct((PAD_ROWS, 128), jnp.float32),
        mesh=_mesh(),
        scratch_types=[
            pltpu.VMEM((2, EMBED, RB), jnp.float32),
            pltpu.VMEM((2, RB, 128), jnp.float32),
            pltpu.SemaphoreType.DMA,
            pltpu.SemaphoreType.DMA,
            pltpu.SemaphoreType.DMA,
            pltpu.SemaphoreType.DMA,
        ],
        compiler_params=_PARAMS,
    )
    tabR = t(embed_table.T)
    if _DEBUG_XLA_GATHER:
        emb = jnp.take(tabR[:NUM_CLASSES, :EMBED], x, axis=0)
        return jnp.transpose(emb, (0, 2, 1))
    g = pl.kernel(
        _gbody,
        out_type=jax.ShapeDtypeStruct((BATCH, EMBED, SEQ), jnp.float32),
        mesh=_mesh(),
        scratch_types=[
            pltpu.VMEM((ROWS_PER_W * SEQ,), jnp.int32),
            pltpu.VMEM((NG, SEQ, 128), jnp.float32),
            pltpu.VMEM((NO, EMBED, SEQ), jnp.float32),
            pltpu.SemaphoreType.DMA,
            pltpu.SemaphoreType.DMA,
            pltpu.SemaphoreType.DMA,
            pltpu.SemaphoreType.DMA,
        ],
        compiler_params=_PARAMS_LINEAR,
    )
    return g(x.reshape(-1), tabR)


def kernel(x, embed_table):
    return _run(x, embed_table)
